# Initial kernel scaffold; baseline (speedup 1.0000x reference)
import jax
import jax.numpy as jnp
from jax.experimental import pallas as pl


def kernel(x, edge_index, W1, W2, Wb1, Wb2):
    raise NotImplementedError("write your pallas kernel here")


# speedup vs baseline: 7.3253x; 7.3253x over previous
import functools

import jax
import jax.numpy as jnp
from jax import lax
from jax.experimental import pallas as pl
from jax.experimental.pallas import tpu as pltpu
from jax.experimental.pallas import tpu_sc as plsc

N = 10000
E = 160000
EPAD = 163840
NROW = 10112
RPT = 632
CH = 128
NCH_AGG = 80
NCH_DEG = 40
D = 256
F32 = jnp.float32

@functools.lru_cache(maxsize=None)
def _mesh():
    return plsc.VectorSubcoreMesh(core_axis_name="c", subcore_axis_name="s",
                                  num_cores=2, num_subcores=16)


def _copy_row(src_mat, row_i, dst_vec):
    for k in range(CH // 16):
        dst_vec[pl.ds(k * 16, 16)] = src_mat[row_i, pl.ds(k * 16, 16)]


def _deg_body(dsts, zrows8, ones8, out, hist, dstmat, dbuf, ones_v):
    c = lax.axis_index("c")
    s = lax.axis_index("s")
    t = c * 16 + s
    pltpu.sync_copy(dsts.at[t], dstmat)
    pltpu.sync_copy(ones8, ones_v)
    r0 = s * RPT
    pltpu.sync_copy(zrows8.at[pl.ds(r0, RPT)], hist.at[pl.ds(r0, RPT)])
    plsc.subcore_barrier()

    @pl.loop(0, NCH_DEG)
    def _(ci):
        _copy_row(dstmat, ci, dbuf)
        pltpu.sync_copy(ones_v, hist.at[dbuf], add=True)

    plsc.subcore_barrier()
    pltpu.sync_copy(hist.at[pl.ds(r0, RPT)], out.at[c, pl.ds(r0, RPT)])


@functools.lru_cache(maxsize=None)
def _deg():
    return pl.kernel(
        _deg_body,
        out_type=jax.ShapeDtypeStruct((2, NROW, 8), F32),
        mesh=_mesh(),
        scratch_types=[
            pltpu.VMEM_SHARED((NROW, 8), F32),
            pltpu.VMEM((NCH_DEG, CH), jnp.int32),
            pltpu.VMEM((CH,), jnp.int32),
            pltpu.VMEM((CH, 8), F32),
        ],
    )


def _agg_body(ycat, srcs, dsts, zrows, out,
              acc, srcmat, dbuf0, dbuf1, rows0, rows1,
              gsem0, gsem1, dsem0, dsem1):
    c = lax.axis_index("c")
    s = lax.axis_index("s")
    pltpu.sync_copy(srcs.at[c, s], srcmat)
    r0 = s * RPT
    pltpu.sync_copy(zrows.at[pl.ds(r0, RPT)], acc.at[pl.ds(r0, RPT)])
    plsc.subcore_barrier()

    rows = (rows0, rows1)
    dbuf = (dbuf0, dbuf1)
    gsem = (gsem0, gsem1)
    dsem = (dsem0, dsem1)

    for b in range(2):
        pltpu.async_copy(dsts.at[s, b], dbuf[b], dsem[b])
        pltpu.async_copy(ycat.at[srcmat.at[b]], rows[b], gsem[b])

    @pl.loop(0, NCH_AGG // 2)
    def _(g):
        for b in range(2):
            ci = 2 * g + b
            pltpu.make_async_copy(ycat.at[srcmat.at[ci]], rows[b], gsem[b]).wait()
            pltpu.make_async_copy(dsts.at[s, ci], dbuf[b], dsem[b]).wait()
            pltpu.sync_copy(rows[b], acc.at[dbuf[b]], add=True)
            nxt = ci + 2

            @pl.when(nxt < NCH_AGG)
            def _():
                pltpu.async_copy(dsts.at[s, nxt], dbuf[b], dsem[b])
                pltpu.async_copy(ycat.at[srcmat.at[nxt]], rows[b], gsem[b])

    plsc.subcore_barrier()
    pltpu.sync_copy(acc.at[pl.ds(r0, RPT)], out.at[c, pl.ds(r0, RPT)])


@functools.lru_cache(maxsize=None)
def _agg():
    return pl.kernel(
        _agg_body,
        out_type=jax.ShapeDtypeStruct((2, NROW, 128), F32),
        mesh=_mesh(),
        scratch_types=[
            pltpu.VMEM_SHARED((NROW, 128), F32),
            pltpu.VMEM((NCH_AGG, CH), jnp.int32),
            pltpu.VMEM((CH,), jnp.int32),
            pltpu.VMEM((CH,), jnp.int32),
            pltpu.VMEM((CH, 128), F32),
            pltpu.VMEM((CH, 128), F32),
            pltpu.SemaphoreType.DMA,
            pltpu.SemaphoreType.DMA,
            pltpu.SemaphoreType.DMA,
            pltpu.SemaphoreType.DMA,
        ],
    )


ROWT = 400
GRID = N // ROWT


def _mm1_body(x_ref, w_ref, o_ref):
    o_ref[...] = lax.dot_general(x_ref[...], w_ref[...],
                                 (((1,), (1,)), ((), ())),
                                 preferred_element_type=F32)


_mm1 = pl.pallas_call(
    _mm1_body,
    grid=(GRID,),
    in_specs=[pl.BlockSpec((ROWT, D), lambda i: (i, 0)),
              pl.BlockSpec((D, D), lambda i: (0, 0))],
    out_specs=pl.BlockSpec((ROWT, D), lambda i: (i, 0)),
    out_shape=jax.ShapeDtypeStruct((N, D), F32),
)


def _scale_body(xw_ref, h8_ref, y_ref, dv_ref):
    hist = h8_ref[0, :, 0] + h8_ref[1, :, 0]
    dv = lax.rsqrt(hist + 1.0)
    dv_ref[...] = dv[:, None]
    y = xw_ref[...] * dv[:, None]
    y_ref[0] = y[:, :128]
    y_ref[1] = y[:, 128:]


_scale = pl.pallas_call(
    _scale_body,
    grid=(GRID,),
    in_specs=[pl.BlockSpec((ROWT, D), lambda i: (i, 0)),
              pl.BlockSpec((2, ROWT, 8), lambda i: (0, i, 0))],
    out_specs=[pl.BlockSpec((2, ROWT, 128), lambda i: (0, i, 0)),
               pl.BlockSpec((ROWT, 1), lambda i: (i, 0))],
    out_shape=[jax.ShapeDtypeStruct((2, N, 128), F32),
               jax.ShapeDtypeStruct((N, 1), F32)],
)


def _bcos1_body(agg_ref, y1_ref, dv_ref, wb_ref, w2_ref, z_ref, s_ref, y2_ref):
    a = jnp.concatenate([agg_ref[0], agg_ref[1]], axis=1)
    yy = jnp.concatenate([y1_ref[0], y1_ref[1]], axis=1)
    dv = dv_ref[...]
    z = dv * (a + yy)
    z_ref[...] = z
    rn = jnp.maximum(jnp.sqrt(jnp.sum(z * z, axis=1, keepdims=True)), 1e-12)
    wb = wb_ref[...]
    wn = jnp.maximum(jnp.sqrt(jnp.sum(wb * wb, axis=1)), 1e-12)
    lin = lax.dot_general(z, wb, (((1,), (1,)), ((), ())),
                          preferred_element_type=F32)
    sc = jnp.maximum(lin / (rn * wn[None, :]), 1e-6)
    s_ref[...] = sc
    h = jnp.maximum(lin * sc, 0.0)
    hw = lax.dot_general(h, w2_ref[...], (((1,), (1,)), ((), ())),
                         preferred_element_type=F32)
    y2 = dv * hw
    y2_ref[0] = y2[:, :128]
    y2_ref[1] = y2[:, 128:]


_bcos1 = pl.pallas_call(
    _bcos1_body,
    grid=(GRID,),
    in_specs=[pl.BlockSpec((2, ROWT, 128), lambda i: (0, i, 0)),
              pl.BlockSpec((2, ROWT, 128), lambda i: (0, i, 0)),
              pl.BlockSpec((ROWT, 1), lambda i: (i, 0)),
              pl.BlockSpec((D, D), lambda i: (0, 0)),
              pl.BlockSpec((D, D), lambda i: (0, 0))],
    out_specs=[pl.BlockSpec((ROWT, D), lambda i: (i, 0)),
               pl.BlockSpec((ROWT, D), lambda i: (i, 0)),
               pl.BlockSpec((2, ROWT, 128), lambda i: (0, i, 0))],
    out_shape=[jax.ShapeDtypeStruct((N, D), F32),
               jax.ShapeDtypeStruct((N, D), F32),
               jax.ShapeDtypeStruct((2, N, 128), F32)],
)


def _bcos2_body(agg_ref, y2_ref, dv_ref, wb_ref, z_ref, s_ref, o_ref):
    a = jnp.concatenate([agg_ref[0], agg_ref[1]], axis=1)
    yy = jnp.concatenate([y2_ref[0], y2_ref[1]], axis=1)
    dv = dv_ref[...]
    z = dv * (a + yy)
    z_ref[...] = z
    rn = jnp.maximum(jnp.sqrt(jnp.sum(z * z, axis=1, keepdims=True)), 1e-12)
    wb = wb_ref[...]
    wn = jnp.maximum(jnp.sqrt(jnp.sum(wb * wb, axis=1)), 1e-12)
    lin = lax.dot_general(z, wb, (((1,), (1,)), ((), ())),
                          preferred_element_type=F32)
    sc = jnp.maximum(lin / (rn * wn[None, :]), 1e-6)
    s_ref[...] = sc
    o_ref[...] = lin * sc


_bcos2 = pl.pallas_call(
    _bcos2_body,
    grid=(GRID,),
    in_specs=[pl.BlockSpec((2, ROWT, 128), lambda i: (0, i, 0)),
              pl.BlockSpec((2, ROWT, 128), lambda i: (0, i, 0)),
              pl.BlockSpec((ROWT, 1), lambda i: (i, 0)),
              pl.BlockSpec((128, D), lambda i: (0, 0))],
    out_specs=[pl.BlockSpec((ROWT, D), lambda i: (i, 0)),
               pl.BlockSpec((ROWT, 128), lambda i: (i, 0)),
               pl.BlockSpec((ROWT, 128), lambda i: (i, 0))],
    out_shape=[jax.ShapeDtypeStruct((N, D), F32),
               jax.ShapeDtypeStruct((N, 128), F32),
               jax.ShapeDtypeStruct((N, 128), F32)],
)


def kernel(x, edge_index, W1, W2, Wb1, Wb2):
    src = edge_index[0].astype(jnp.int32)
    dst = edge_index[1].astype(jnp.int32)
    pad_s = jnp.zeros((EPAD - E,), jnp.int32)
    pad_d = jnp.full((EPAD - E,), N, jnp.int32)
    src_p = jnp.concatenate([src, pad_s])
    dst_p = jnp.concatenate([dst, pad_d])
    srcs2 = jnp.stack([src_p, src_p + N]).reshape(2, 16, NCH_AGG, CH)
    dst_agg = dst_p.reshape(16, NCH_AGG, CH)
    dst_deg = dst_p.reshape(32, NCH_DEG, CH)
    zrows = jnp.zeros((NROW, 128), F32)
    zrows8 = jnp.zeros((NROW, 8), F32)
    ones8 = jnp.concatenate(
        [jnp.ones((CH, 1), F32), jnp.zeros((CH, 7), F32)], axis=1)

    hist8 = _deg()(dst_deg, zrows8, ones8)
    xw = _mm1(x, W1)
    y1, dinv = _scale(xw, hist8)
    agg1 = _agg()(y1.reshape(2 * N, 128), srcs2, dst_agg, zrows)
    z1, s1, y2 = _bcos1(agg1, y1, dinv, Wb1, W2)
    agg2 = _agg()(y2.reshape(2 * N, 128), srcs2, dst_agg, zrows)
    z2, s2, out = _bcos2(agg2, y2, dinv, Wb2)
    return (out, z1, z2, s1, s2)
